# baseline (speedup 1.0000x reference)
import functools

import jax
import jax.numpy as jnp
from jax import lax
from jax.experimental import pallas as pl
from jax.experimental.pallas import tpu as pltpu
from jax.experimental.pallas import tpu_sc as plsc

_B, _S, _D = 4, 4096, 2048
_NC, _NS = 2, 16
_NW = _NC * _NS
_RPW = (_B * _S) // _NW
_G = 16
_NGRP = _RPW // _G
_LAG = 4

_mesh = plsc.VectorSubcoreMesh(core_axis_name="c", subcore_axis_name="s")


@functools.partial(
    pl.kernel,
    mesh=_mesh,
    out_type=jax.ShapeDtypeStruct((_B * _S, _D), jnp.float32),
    scratch_types=[
        pltpu.VMEM((_RPW,), jnp.int32),
        pltpu.SemaphoreType.DMA,
    ],
)
def _permute_rows(x_hbm, gidx_hbm, out_hbm, idx_v, sem):
    wid = lax.axis_index("s") * _NC + lax.axis_index("c")
    base = wid * _RPW
    pltpu.sync_copy(gidx_hbm.at[pl.ds(base, _RPW)], idx_v)

    def drain(nrows):
        pltpu.make_async_copy(x_hbm.at[pl.ds(0, nrows)],
                              out_hbm.at[pl.ds(base, nrows)], sem).wait()

    def group(g, carry):
        vec = idx_v[pl.ds(g * _G, _G)]
        gbase = base + g * _G
        for j in range(_G):
            pltpu.async_copy(x_hbm.at[pl.ds(vec[j], 1)],
                             out_hbm.at[pl.ds(gbase + j, 1)], sem).start()

        @pl.when(g >= _LAG)
        def _():
            drain(_G)
        return carry

    lax.fori_loop(0, _NGRP, group, 0)
    drain(_LAG * _G)


def kernel(x, perm_indices):
    gidx = (perm_indices.astype(jnp.int32).reshape(1, _S)
            + (jnp.arange(_B, dtype=jnp.int32) * _S).reshape(_B, 1)).reshape(-1)
    out = _permute_rows(x.reshape(_B * _S, _D), gidx)
    return out.reshape(_B, _S, _D)


# speedup vs baseline: 72.4344x; 72.4344x over previous
import functools

import jax
import jax.numpy as jnp
from jax import lax
from jax.experimental import pallas as pl
from jax.experimental.pallas import tpu as pltpu
from jax.experimental.pallas import tpu_sc as plsc

_B, _S, _D = 4, 4096, 2048
_NC, _NS = 2, 16
_NW = _NC * _NS
_RPW = (_B * _S) // _NW
_K = 16
_NCHUNK = _RPW // _K
_NPAIR = _NCHUNK // 2

_mesh = plsc.VectorSubcoreMesh(core_axis_name="c", subcore_axis_name="s")


@functools.partial(
    pl.kernel,
    mesh=_mesh,
    out_type=jax.ShapeDtypeStruct((_B * _S, _D), jnp.float32),
    scratch_types=[
        pltpu.VMEM((_RPW,), jnp.int32),
        pltpu.VMEM((_K, _D), jnp.float32),
        pltpu.VMEM((_K, _D), jnp.float32),
        pltpu.SemaphoreType.DMA,
        pltpu.SemaphoreType.DMA,
        pltpu.SemaphoreType.DMA,
        pltpu.SemaphoreType.DMA,
    ],
)
def _permute_rows(x_hbm, gidx_hbm, out_hbm, idx_v, buf0, buf1, g0, g1, s0, s1):
    wid = lax.axis_index("s") * _NC + lax.axis_index("c")
    base = wid * _RPW
    pltpu.sync_copy(gidx_hbm.at[pl.ds(base, _RPW)], idx_v)

    def gather(c, buf, sem):
        return pltpu.make_async_copy(
            x_hbm.at[idx_v.at[pl.ds(c * _K, _K)]], buf, sem)

    def store(c, buf, sem):
        return pltpu.make_async_copy(
            buf, out_hbm.at[pl.ds(base + c * _K, _K)], sem)

    gather(0, buf0, g0).start()

    def body(i, carry):
        c0 = 2 * i
        c1 = c0 + 1

        @pl.when(i > 0)
        def _():
            store(c1 - 2, buf1, s1).wait()
        gather(c1, buf1, g1).start()

        gather(c0, buf0, g0).wait()
        store(c0, buf0, s0).start()

        @pl.when(i < _NPAIR - 1)
        def _():
            store(c0, buf0, s0).wait()
            gather(c0 + 2, buf0, g0).start()

        gather(c1, buf1, g1).wait()
        store(c1, buf1, s1).start()
        return carry

    lax.fori_loop(0, _NPAIR, body, 0)

    store(_NCHUNK - 2, buf0, s0).wait()
    store(_NCHUNK - 1, buf1, s1).wait()


def kernel(x, perm_indices):
    gidx = (perm_indices.astype(jnp.int32).reshape(1, _S)
            + (jnp.arange(_B, dtype=jnp.int32) * _S).reshape(_B, 1)).reshape(-1)
    out = _permute_rows(x.reshape(_B * _S, _D), gidx)
    return out.reshape(_B, _S, _D)


# speedup vs baseline: 72.6606x; 1.0031x over previous
import functools

import jax
import jax.numpy as jnp
from jax import lax
from jax.experimental import pallas as pl
from jax.experimental.pallas import tpu as pltpu
from jax.experimental.pallas import tpu_sc as plsc

_B, _S, _D = 4, 4096, 2048
_NC, _NS = 2, 16
_NW = _NC * _NS
_RPW = (_B * _S) // _NW
_K = 8
_NBUF = 4
_NCHUNK = _RPW // _K
_NITER = _NCHUNK // _NBUF

_mesh = plsc.VectorSubcoreMesh(core_axis_name="c", subcore_axis_name="s")


@functools.partial(
    pl.kernel,
    mesh=_mesh,
    out_type=jax.ShapeDtypeStruct((_B * _S, _D), jnp.float32),
    scratch_types=(
        [pltpu.VMEM((_RPW,), jnp.int32)]
        + [pltpu.VMEM((_K, _D), jnp.float32)] * _NBUF
        + [pltpu.SemaphoreType.DMA] * (2 * _NBUF)
    ),
)
def _permute_rows(x_hbm, gidx_hbm, out_hbm, idx_v, *rest):
    bufs = rest[:_NBUF]
    gsem = rest[_NBUF:2 * _NBUF]
    ssem = rest[2 * _NBUF:]

    wid = lax.axis_index("s") * _NC + lax.axis_index("c")
    base = wid * _RPW
    pltpu.sync_copy(gidx_hbm.at[pl.ds(base, _RPW)], idx_v)

    def gather(c, j):
        return pltpu.make_async_copy(
            x_hbm.at[idx_v.at[pl.ds(c * _K, _K)]], bufs[j], gsem[j])

    def store(c, j):
        return pltpu.make_async_copy(
            bufs[j], out_hbm.at[pl.ds(base + c * _K, _K)], ssem[j])

    for j in range(_NBUF):
        gather(j, j).start()

    def body(i, carry):
        for j in range(_NBUF):
            c = i * _NBUF + j
            gather(c, j).wait()
            store(c, j).start()

            @pl.when(i < _NITER - 1)
            def _():
                store(c, j).wait()
                gather(c + _NBUF, j).start()
        return carry

    lax.fori_loop(0, _NITER, body, 0)

    for j in range(_NBUF):
        store(_NCHUNK - _NBUF + j, j).wait()


def kernel(x, perm_indices):
    gidx = (perm_indices.astype(jnp.int32).reshape(1, _S)
            + (jnp.arange(_B, dtype=jnp.int32) * _S).reshape(_B, 1)).reshape(-1)
    out = _permute_rows(x.reshape(_B * _S, _D), gidx)
    return out.reshape(_B, _S, _D)
